# baseline (speedup 1.0000x reference)
import functools

import jax
import jax.numpy as jnp
from jax import lax
from jax.experimental import pallas as pl
from jax.experimental.pallas import tpu as pltpu
from jax.experimental.pallas import tpu_sc as plsc

_NC = 2
_NS = 16
_LANES = 128
_K = 6
_UNROLL = 6
_F = 16
_PK = 8


def _cdiv(a, b):
    return -(-a // b)


@functools.lru_cache(maxsize=None)
def _seg_make(n_pad, rpt_e, rpt_n, feat):
    f32 = jnp.float32
    mesh = plsc.VectorSubcoreMesh(core_axis_name="c", subcore_axis_name="s",
                                  num_cores=_NC, num_subcores=_NS)
    cpt = rpt_e // _K

    def body(hn, src2, dst2, zz, p0, p1, acc,
             src_a, src_b, dst_a, dst_b, dst_c, rows_a, rows_b,
             lsem, gsem, ssem):
        cid = lax.axis_index("c")
        sid = lax.axis_index("s")
        srcs = (src_a, src_b)
        dsts = (dst_a, dst_b, dst_c)
        rows = (rows_a, rows_b)
        nsl = pl.ds(sid * rpt_n, rpt_n)
        pltpu.sync_copy(zz.at[nsl], acc.at[nsl])
        plsc.subcore_barrier()
        row0 = (cid * _NS + sid) * rpt_e

        def fire_idx(g, b2, b3):
            base = row0 + g * _K
            pltpu.async_copy(src2.at[pl.ds(base, _K)], srcs[b2], lsem)
            pltpu.async_copy(dst2.at[pl.ds(base, _K)], dsts[b3], lsem)

        def drain_scat(b2):
            for j in range(_K):
                pltpu.make_async_copy(zz.at[pl.ds(0, _LANES)],
                                      rows[b2].at[j], ssem).wait()

        def one_chunk(g, u):
            b2, b3 = u % 2, u % 3

            @pl.when(g >= 2)
            def _():
                drain_scat(b2)
            pltpu.make_async_copy(src2.at[pl.ds(0, _K)], srcs[b2], lsem).wait()
            pltpu.make_async_copy(src2.at[pl.ds(0, _K)], dsts[b3], lsem).wait()

            @pl.when(g + 1 < cpt)
            def _():
                fire_idx(g + 1, (u + 1) % 2, (u + 1) % 3)

            gd = [pltpu.async_copy(hn.at[srcs[b2].at[j]], rows[b2].at[j], gsem)
                  for j in range(_K)]
            for j in range(_K):
                gd[j].wait()
            for j in range(_K):
                pltpu.async_copy(rows[b2].at[j], acc.at[dsts[b3].at[j]],
                                 ssem, add=True)

        fire_idx(0, 0, 0)

        def step(gs, carry):
            for u in range(_UNROLL):
                one_chunk(gs * _UNROLL + u, u)
            return carry

        lax.fori_loop(0, cpt // _UNROLL, step, 0)
        for u in range(2):
            drain_scat((cpt - 2 + u) % 2)
        plsc.subcore_barrier()

        @pl.when(cid == 0)
        def _():
            pltpu.sync_copy(acc.at[nsl], p0.at[nsl])

        @pl.when(cid == 1)
        def _():
            pltpu.sync_copy(acc.at[nsl], p1.at[nsl])

    return pl.kernel(
        body,
        out_type=(jax.ShapeDtypeStruct((n_pad, feat), f32),
                  jax.ShapeDtypeStruct((n_pad, feat), f32)),
        mesh=mesh,
        scratch_types=(pltpu.VMEM_SHARED((n_pad, feat), f32),
                       pltpu.VMEM((_K, _LANES), jnp.int32),
                       pltpu.VMEM((_K, _LANES), jnp.int32),
                       pltpu.VMEM((_K, _LANES), jnp.int32),
                       pltpu.VMEM((_K, _LANES), jnp.int32),
                       pltpu.VMEM((_K, _LANES), jnp.int32),
                       pltpu.VMEM((_K, _LANES, feat), f32),
                       pltpu.VMEM((_K, _LANES, feat), f32),
                       pltpu.SemaphoreType.DMA,
                       pltpu.SemaphoreType.DMA,
                       pltpu.SemaphoreType.DMA),
        compiler_params=pltpu.CompilerParams(use_tc_tiling_on_sc=False))


@functools.lru_cache(maxsize=None)
def _deg_make(n_pad, rpt_e, rpt_n, feat):
    f32 = jnp.float32
    mesh = plsc.VectorSubcoreMesh(core_axis_name="c", subcore_axis_name="s",
                                  num_cores=_NC, num_subcores=_NS)
    cpt = rpt_e // _K

    def body(dst2, zz, ones, q0, q1, accd, dst_a, dst_b, dst_c, ones_v,
             lsem, ssem):
        cid = lax.axis_index("c")
        sid = lax.axis_index("s")
        dsts = (dst_a, dst_b, dst_c)
        nsl = pl.ds(sid * rpt_n, rpt_n)
        pltpu.sync_copy(zz.at[nsl], accd.at[nsl])
        pltpu.sync_copy(ones, ones_v)
        plsc.subcore_barrier()
        row0 = (cid * _NS + sid) * rpt_e

        def fire_idx(g, b3):
            base = row0 + g * _K
            pltpu.async_copy(dst2.at[pl.ds(base, _K)], dsts[b3], lsem)

        def drain_scat():
            for j in range(_K):
                pltpu.make_async_copy(zz.at[pl.ds(0, _LANES)],
                                      ones_v, ssem).wait()

        def one_chunk(g, u):
            b3 = u % 3

            @pl.when(g >= 2)
            def _():
                drain_scat()

            pltpu.make_async_copy(dst2.at[pl.ds(0, _K)], dsts[b3], lsem).wait()

            @pl.when(g + 1 < cpt)
            def _():
                fire_idx(g + 1, (u + 1) % 3)

            for j in range(_K):
                pltpu.async_copy(ones_v, accd.at[dsts[b3].at[j]],
                                 ssem, add=True)

        fire_idx(0, 0)

        def step(gs, carry):
            for u in range(_UNROLL):
                one_chunk(gs * _UNROLL + u, u)
            return carry

        lax.fori_loop(0, cpt // _UNROLL, step, 0)
        for _u in range(2):
            drain_scat()
        plsc.subcore_barrier()

        @pl.when(cid == 0)
        def _():
            pltpu.sync_copy(accd.at[nsl], q0.at[nsl])

        @pl.when(cid == 1)
        def _():
            pltpu.sync_copy(accd.at[nsl], q1.at[nsl])

    return pl.kernel(
        body,
        out_type=(jax.ShapeDtypeStruct((n_pad, feat), f32),
                  jax.ShapeDtypeStruct((n_pad, feat), f32)),
        mesh=mesh,
        scratch_types=(pltpu.VMEM_SHARED((n_pad, feat), f32),
                       pltpu.VMEM((_K, _LANES), jnp.int32),
                       pltpu.VMEM((_K, _LANES), jnp.int32),
                       pltpu.VMEM((_K, _LANES), jnp.int32),
                       pltpu.VMEM((_LANES, feat), f32),
                       pltpu.SemaphoreType.DMA,
                       pltpu.SemaphoreType.DMA),
        compiler_params=pltpu.CompilerParams(use_tc_tiling_on_sc=False))


def _prep_body(x_ref, w_ref, q0_ref, q1_ref, hn_ref, d_ref):
    d = lax.rsqrt(q0_ref[...] + q1_ref[...] + 1.0)
    d_ref[...] = d
    hn_ref[...] = jnp.dot(x_ref[...], w_ref[...],
                          preferred_element_type=jnp.float32) * d


def _mid_body(p0_ref, p1_ref, hn_ref, d_ref, b_ref, w_ref, o_ref):
    d = d_ref[...]
    t = (p0_ref[...] + p1_ref[...] + hn_ref[...]) * d + b_ref[...]
    t = jnp.maximum(t, 0.0)
    o_ref[...] = jnp.dot(t, w_ref[...], preferred_element_type=jnp.float32) * d


def _fin_body(p0_ref, p1_ref, hn_ref, d_ref, b_ref, o_ref):
    o_ref[...] = (p0_ref[...] + p1_ref[...] + hn_ref[...]) * d_ref[...] + b_ref[...]


def _row_spec(blk):
    return pl.BlockSpec((blk, _PK * _F), lambda i: (i, 0))


def _full_spec(shape):
    return pl.BlockSpec(shape, lambda i: (0, 0))


def _tc_call(body, rows_pk, in_arrays, in_specs, n_out):
    blk = rows_pk // 4
    oshape = jax.ShapeDtypeStruct((rows_pk, _PK * _F), jnp.float32)
    out_shape = [oshape] * n_out if n_out > 1 else oshape
    out_specs = [_row_spec(blk)] * n_out if n_out > 1 else _row_spec(blk)
    return pl.pallas_call(
        body,
        grid=(4,),
        in_specs=in_specs,
        out_specs=out_specs,
        out_shape=out_shape)(*in_arrays)


def kernel(x, edge_index, batch_index, W1, b1, W2, b2, W3, b3):
    f32 = jnp.float32
    n, seq = x.shape
    e = edge_index.shape[1]
    emb = W1.shape[1]
    out_d = W3.shape[1]

    n_pad = _cdiv(n + 1, 1024) * 1024
    rpt_n = n_pad // _NS
    rows_pk = n_pad // _PK
    cpt = _cdiv(_cdiv(e, _NC * _NS * _K * _LANES), _UNROLL) * _UNROLL
    rpt_e = cpt * _K
    rows2d = rpt_e * _NC * _NS
    pad = rows2d * _LANES - e

    src2 = jnp.concatenate(
        [edge_index[0], jnp.zeros((pad,), jnp.int32)]).reshape(rows2d, _LANES)
    dst2 = jnp.concatenate(
        [edge_index[1], jnp.full((pad,), n, jnp.int32)]).reshape(rows2d, _LANES)

    eye8 = jnp.eye(_PK, dtype=f32)
    xp = jnp.pad(x, ((0, n_pad - n), (0, _F - seq))).reshape(rows_pk, _PK * _F)
    W1b = jnp.kron(eye8, jnp.pad(W1, ((0, _F - seq), (0, _F - emb))))
    W2b = jnp.kron(eye8, jnp.pad(W2, ((0, _F - emb), (0, _F - emb))))
    W3b = jnp.kron(eye8, jnp.pad(W3, ((0, _F - emb), (0, _F - out_d))))
    b1b = jnp.tile(jnp.pad(b1, (0, _F - emb)), _PK).reshape(1, _PK * _F)
    b2b = jnp.tile(jnp.pad(b2, (0, _F - emb)), _PK).reshape(1, _PK * _F)
    b3b = jnp.tile(jnp.pad(b3, (0, _F - out_d)), _PK).reshape(1, _PK * _F)

    zz = jnp.zeros((n_pad, _F), f32)
    ones = jnp.ones((_LANES, _F), f32)

    deg_fn = _deg_make(n_pad, rpt_e, rpt_n, _F)
    seg_fn = _seg_make(n_pad, rpt_e, rpt_n, _F)

    def pk(a):
        return a.reshape(rows_pk, _PK * _F)

    def unpk(a):
        return a.reshape(n_pad, _F)

    dq0, dq1 = deg_fn(dst2, zz, ones)

    hn1, dpk = _tc_call(_prep_body, rows_pk, (xp, W1b, pk(dq0), pk(dq1)),
                        [_row_spec(rows_pk // 4), _full_spec((_PK * _F, _PK * _F)),
                         _row_spec(rows_pk // 4), _row_spec(rows_pk // 4)], 2)

    s0, s1 = seg_fn(unpk(hn1), src2, dst2, zz)
    hn2 = _tc_call(_mid_body, rows_pk, (pk(s0), pk(s1), hn1, dpk, b1b, W2b),
                   [_row_spec(rows_pk // 4)] * 4 +
                   [_full_spec((1, _PK * _F)), _full_spec((_PK * _F, _PK * _F))], 1)

    s0, s1 = seg_fn(unpk(hn2), src2, dst2, zz)
    hn3 = _tc_call(_mid_body, rows_pk, (pk(s0), pk(s1), hn2, dpk, b2b, W3b),
                   [_row_spec(rows_pk // 4)] * 4 +
                   [_full_spec((1, _PK * _F)), _full_spec((_PK * _F, _PK * _F))], 1)

    s0, s1 = seg_fn(unpk(hn3), src2, dst2, zz)
    outp = _tc_call(_fin_body, rows_pk, (pk(s0), pk(s1), hn3, dpk, b3b),
                    [_row_spec(rows_pk // 4)] * 4 +
                    [_full_spec((1, _PK * _F))], 1)

    return unpk(outp)[:n, :out_d]


# speedup vs baseline: 1.1125x; 1.1125x over previous
import functools

import jax
import jax.numpy as jnp
from jax import lax
from jax.experimental import pallas as pl
from jax.experimental.pallas import tpu as pltpu
from jax.experimental.pallas import tpu_sc as plsc

_NC = 2
_NS = 16
_LANES = 128
_K = 6
_UNROLL = 6
_F = 16
_PK = 8


def _cdiv(a, b):
    return -(-a // b)


@functools.lru_cache(maxsize=None)
def _seg_make(n_pad, rpt_n, feat, cpt0, cpt1):
    f32 = jnp.float32
    mesh = plsc.VectorSubcoreMesh(core_axis_name="c", subcore_axis_name="s",
                                  num_cores=_NC, num_subcores=_NS)
    rpt_e0, rpt_e1 = cpt0 * _K, cpt1 * _K

    def body(hn, src2, dst2, zz, p0, p1, acc,
             src_a, src_b, dst_a, dst_b, dst_c, rows_a, rows_b,
             lsem, gsem, ssem):
        cid = lax.axis_index("c")
        sid = lax.axis_index("s")
        srcs = (src_a, src_b)
        dsts = (dst_a, dst_b, dst_c)
        rows = (rows_a, rows_b)
        nsl = pl.ds(sid * rpt_n, rpt_n)
        pltpu.sync_copy(zz.at[nsl], acc.at[nsl])
        plsc.subcore_barrier()
        row0 = jnp.where(cid == 0, sid * rpt_e0,
                         _NS * rpt_e0 + sid * rpt_e1)
        cpt = jnp.where(cid == 0, cpt0, cpt1)

        def fire_idx(g, b2, b3):
            base = row0 + g * _K
            pltpu.async_copy(src2.at[pl.ds(base, _K)], srcs[b2], lsem)
            pltpu.async_copy(dst2.at[pl.ds(base, _K)], dsts[b3], lsem)

        def drain_scat(b2):
            for j in range(_K):
                pltpu.make_async_copy(zz.at[pl.ds(0, _LANES)],
                                      rows[b2].at[j], ssem).wait()

        def one_chunk(g, u):
            b2, b3 = u % 2, u % 3

            @pl.when(g >= 2)
            def _():
                drain_scat(b2)
            pltpu.make_async_copy(src2.at[pl.ds(0, _K)], srcs[b2], lsem).wait()
            pltpu.make_async_copy(src2.at[pl.ds(0, _K)], dsts[b3], lsem).wait()

            @pl.when(g + 1 < cpt)
            def _():
                fire_idx(g + 1, (u + 1) % 2, (u + 1) % 3)

            gd = [pltpu.async_copy(hn.at[srcs[b2].at[j]], rows[b2].at[j], gsem)
                  for j in range(_K)]
            for j in range(_K):
                gd[j].wait()
            for j in range(_K):
                pltpu.async_copy(rows[b2].at[j], acc.at[dsts[b3].at[j]],
                                 ssem, add=True)

        fire_idx(0, 0, 0)

        def step(gs, carry):
            for u in range(_UNROLL):
                one_chunk(gs * _UNROLL + u, u)
            return carry

        lax.fori_loop(0, cpt // _UNROLL, step, 0)
        for u in range(2):
            drain_scat(u)
        plsc.subcore_barrier()

        @pl.when(cid == 0)
        def _():
            pltpu.sync_copy(acc.at[nsl], p0.at[nsl])

        @pl.when(cid == 1)
        def _():
            pltpu.sync_copy(acc.at[nsl], p1.at[nsl])

    return pl.kernel(
        body,
        out_type=(jax.ShapeDtypeStruct((n_pad, feat), f32),
                  jax.ShapeDtypeStruct((n_pad, feat), f32)),
        mesh=mesh,
        scratch_types=(pltpu.VMEM_SHARED((n_pad, feat), f32),
                       pltpu.VMEM((_K, _LANES), jnp.int32),
                       pltpu.VMEM((_K, _LANES), jnp.int32),
                       pltpu.VMEM((_K, _LANES), jnp.int32),
                       pltpu.VMEM((_K, _LANES), jnp.int32),
                       pltpu.VMEM((_K, _LANES), jnp.int32),
                       pltpu.VMEM((_K, _LANES, feat), f32),
                       pltpu.VMEM((_K, _LANES, feat), f32),
                       pltpu.SemaphoreType.DMA,
                       pltpu.SemaphoreType.DMA,
                       pltpu.SemaphoreType.DMA),
        compiler_params=pltpu.CompilerParams(use_tc_tiling_on_sc=False))


@functools.lru_cache(maxsize=None)
def _deg_make(n_pad, rpt_n, feat, cpt0, cpt1):
    f32 = jnp.float32
    mesh = plsc.VectorSubcoreMesh(core_axis_name="c", subcore_axis_name="s",
                                  num_cores=_NC, num_subcores=_NS)
    rpt_e0, rpt_e1 = cpt0 * _K, cpt1 * _K

    def body(dst2, zz, ones, q0, q1, accd, dst_a, dst_b, dst_c, ones_v,
             lsem, ssem):
        cid = lax.axis_index("c")
        sid = lax.axis_index("s")
        dsts = (dst_a, dst_b, dst_c)
        nsl = pl.ds(sid * rpt_n, rpt_n)
        pltpu.sync_copy(zz.at[nsl], accd.at[nsl])
        pltpu.sync_copy(ones, ones_v)
        plsc.subcore_barrier()
        row0 = jnp.where(cid == 0, sid * rpt_e0,
                         _NS * rpt_e0 + sid * rpt_e1)
        cpt = jnp.where(cid == 0, cpt0, cpt1)

        def fire_idx(g, b3):
            base = row0 + g * _K
            pltpu.async_copy(dst2.at[pl.ds(base, _K)], dsts[b3], lsem)

        def drain_scat():
            for j in range(_K):
                pltpu.make_async_copy(zz.at[pl.ds(0, _LANES)],
                                      ones_v, ssem).wait()

        def one_chunk(g, u):
            b3 = u % 3

            @pl.when(g >= 2)
            def _():
                drain_scat()

            pltpu.make_async_copy(dst2.at[pl.ds(0, _K)], dsts[b3], lsem).wait()

            @pl.when(g + 1 < cpt)
            def _():
                fire_idx(g + 1, (u + 1) % 3)

            for j in range(_K):
                pltpu.async_copy(ones_v, accd.at[dsts[b3].at[j]],
                                 ssem, add=True)

        fire_idx(0, 0)

        def step(gs, carry):
            for u in range(_UNROLL):
                one_chunk(gs * _UNROLL + u, u)
            return carry

        lax.fori_loop(0, cpt // _UNROLL, step, 0)
        for _u in range(2):
            drain_scat()
        plsc.subcore_barrier()

        @pl.when(cid == 0)
        def _():
            pltpu.sync_copy(accd.at[nsl], q0.at[nsl])

        @pl.when(cid == 1)
        def _():
            pltpu.sync_copy(accd.at[nsl], q1.at[nsl])

    return pl.kernel(
        body,
        out_type=(jax.ShapeDtypeStruct((n_pad, feat), f32),
                  jax.ShapeDtypeStruct((n_pad, feat), f32)),
        mesh=mesh,
        scratch_types=(pltpu.VMEM_SHARED((n_pad, feat), f32),
                       pltpu.VMEM((_K, _LANES), jnp.int32),
                       pltpu.VMEM((_K, _LANES), jnp.int32),
                       pltpu.VMEM((_K, _LANES), jnp.int32),
                       pltpu.VMEM((_LANES, feat), f32),
                       pltpu.SemaphoreType.DMA,
                       pltpu.SemaphoreType.DMA),
        compiler_params=pltpu.CompilerParams(use_tc_tiling_on_sc=False))


def _prep_body(x_ref, w_ref, q0_ref, q1_ref, hn_ref, d_ref):
    d = lax.rsqrt(q0_ref[...] + q1_ref[...] + 1.0)
    d_ref[...] = d
    hn_ref[...] = jnp.dot(x_ref[...], w_ref[...],
                          preferred_element_type=jnp.float32) * d


def _mid_body(p0_ref, p1_ref, hn_ref, d_ref, b_ref, w_ref, o_ref):
    d = d_ref[...]
    t = (p0_ref[...] + p1_ref[...] + hn_ref[...]) * d + b_ref[...]
    t = jnp.maximum(t, 0.0)
    o_ref[...] = jnp.dot(t, w_ref[...], preferred_element_type=jnp.float32) * d


def _fin_body(p0_ref, p1_ref, hn_ref, d_ref, b_ref, o_ref):
    o_ref[...] = (p0_ref[...] + p1_ref[...] + hn_ref[...]) * d_ref[...] + b_ref[...]


def _row_spec(blk):
    return pl.BlockSpec((blk, _PK * _F), lambda i: (i, 0))


def _full_spec(shape):
    return pl.BlockSpec(shape, lambda i: (0, 0))


def _tc_call(body, rows_pk, in_arrays, in_specs, n_out):
    blk = rows_pk // 4
    oshape = jax.ShapeDtypeStruct((rows_pk, _PK * _F), jnp.float32)
    out_shape = [oshape] * n_out if n_out > 1 else oshape
    out_specs = [_row_spec(blk)] * n_out if n_out > 1 else _row_spec(blk)
    return pl.pallas_call(
        body,
        grid=(4,),
        in_specs=in_specs,
        out_specs=out_specs,
        out_shape=out_shape)(*in_arrays)


def kernel(x, edge_index, batch_index, W1, b1, W2, b2, W3, b3):
    f32 = jnp.float32
    n, seq = x.shape
    e = edge_index.shape[1]
    emb = W1.shape[1]
    out_d = W3.shape[1]

    n_pad = _cdiv(n + 1, 1024) * 1024
    rpt_n = n_pad // _NS
    rows_pk = n_pad // _PK
    ct = _cdiv(_cdiv(e, _NS * _K * _LANES), 2 * _UNROLL) * 2 * _UNROLL
    seg_c0 = int(round(ct * 0.66 / _UNROLL)) * _UNROLL
    deg_c0 = int(round(ct * 0.57 / _UNROLL)) * _UNROLL
    rows2d = _NS * _K * ct
    pad = rows2d * _LANES - e

    src2 = jnp.concatenate(
        [edge_index[0], jnp.zeros((pad,), jnp.int32)]).reshape(rows2d, _LANES)
    dst2 = jnp.concatenate(
        [edge_index[1], jnp.full((pad,), n, jnp.int32)]).reshape(rows2d, _LANES)

    eye8 = jnp.eye(_PK, dtype=f32)
    xp = jnp.pad(x, ((0, n_pad - n), (0, _F - seq))).reshape(rows_pk, _PK * _F)
    W1b = jnp.kron(eye8, jnp.pad(W1, ((0, _F - seq), (0, _F - emb))))
    W2b = jnp.kron(eye8, jnp.pad(W2, ((0, _F - emb), (0, _F - emb))))
    W3b = jnp.kron(eye8, jnp.pad(W3, ((0, _F - emb), (0, _F - out_d))))
    b1b = jnp.tile(jnp.pad(b1, (0, _F - emb)), _PK).reshape(1, _PK * _F)
    b2b = jnp.tile(jnp.pad(b2, (0, _F - emb)), _PK).reshape(1, _PK * _F)
    b3b = jnp.tile(jnp.pad(b3, (0, _F - out_d)), _PK).reshape(1, _PK * _F)

    zz = jnp.zeros((n_pad, _F), f32)
    ones = jnp.ones((_LANES, _F), f32)

    deg_fn = _deg_make(n_pad, rpt_n, _F, deg_c0, ct - deg_c0)
    seg_fn = _seg_make(n_pad, rpt_n, _F, seg_c0, ct - seg_c0)

    def pk(a):
        return a.reshape(rows_pk, _PK * _F)

    def unpk(a):
        return a.reshape(n_pad, _F)

    dq0, dq1 = deg_fn(dst2, zz, ones)

    hn1, dpk = _tc_call(_prep_body, rows_pk, (xp, W1b, pk(dq0), pk(dq1)),
                        [_row_spec(rows_pk // 4), _full_spec((_PK * _F, _PK * _F)),
                         _row_spec(rows_pk // 4), _row_spec(rows_pk // 4)], 2)

    s0, s1 = seg_fn(unpk(hn1), src2, dst2, zz)
    hn2 = _tc_call(_mid_body, rows_pk, (pk(s0), pk(s1), hn1, dpk, b1b, W2b),
                   [_row_spec(rows_pk // 4)] * 4 +
                   [_full_spec((1, _PK * _F)), _full_spec((_PK * _F, _PK * _F))], 1)

    s0, s1 = seg_fn(unpk(hn2), src2, dst2, zz)
    hn3 = _tc_call(_mid_body, rows_pk, (pk(s0), pk(s1), hn2, dpk, b2b, W3b),
                   [_row_spec(rows_pk // 4)] * 4 +
                   [_full_spec((1, _PK * _F)), _full_spec((_PK * _F, _PK * _F))], 1)

    s0, s1 = seg_fn(unpk(hn3), src2, dst2, zz)
    outp = _tc_call(_fin_body, rows_pk, (pk(s0), pk(s1), hn3, dpk, b3b),
                    [_row_spec(rows_pk // 4)] * 4 +
                    [_full_spec((1, _PK * _F))], 1)

    return unpk(outp)[:n, :out_d]


# speedup vs baseline: 1.1684x; 1.0503x over previous
import functools

import jax
import jax.numpy as jnp
from jax import lax
from jax.experimental import pallas as pl
from jax.experimental.pallas import tpu as pltpu
from jax.experimental.pallas import tpu_sc as plsc

_NC = 2
_NS = 16
_LANES = 128
_K = 6
_UNROLL = 6
_F = 16
_PK = 8


def _cdiv(a, b):
    return -(-a // b)


@functools.lru_cache(maxsize=None)
def _seg_make(n_pad, rpt_n, feat, cpt0, cpt1):
    f32 = jnp.float32
    mesh = plsc.VectorSubcoreMesh(core_axis_name="c", subcore_axis_name="s",
                                  num_cores=_NC, num_subcores=_NS)
    rpt_e0, rpt_e1 = cpt0 * _K, cpt1 * _K

    def body(hn, src2, dst2, zz, p0, p1, acc,
             src_a, src_b, dst_a, dst_b, dst_c, rows_a, rows_b,
             lsem, gsem, ssem):
        cid = lax.axis_index("c")
        sid = lax.axis_index("s")
        srcs = (src_a, src_b)
        dsts = (dst_a, dst_b, dst_c)
        rows = (rows_a, rows_b)
        nsl = pl.ds(sid * rpt_n, rpt_n)
        pltpu.sync_copy(zz.at[nsl], acc.at[nsl])
        plsc.subcore_barrier()
        row0 = jnp.where(cid == 0, sid * rpt_e0,
                         _NS * rpt_e0 + sid * rpt_e1)
        cpt = jnp.where(cid == 0, cpt0, cpt1)

        def fire_idx(g, b2, b3):
            base = row0 + g * _K
            pltpu.async_copy(src2.at[pl.ds(base, _K)], srcs[b2], lsem)
            pltpu.async_copy(dst2.at[pl.ds(base, _K)], dsts[b3], lsem)

        def drain_scat(b2):
            for j in range(_K):
                pltpu.make_async_copy(zz.at[pl.ds(0, _LANES)],
                                      rows[b2].at[j], ssem).wait()

        def one_chunk(g, u):
            b2, b3 = u % 2, u % 3

            @pl.when(g >= 2)
            def _():
                drain_scat(b2)
            pltpu.make_async_copy(src2.at[pl.ds(0, _K)], srcs[b2], lsem).wait()
            pltpu.make_async_copy(src2.at[pl.ds(0, _K)], dsts[b3], lsem).wait()

            @pl.when(g + 1 < cpt)
            def _():
                fire_idx(g + 1, (u + 1) % 2, (u + 1) % 3)

            gd = [pltpu.async_copy(hn.at[srcs[b2].at[j]], rows[b2].at[j], gsem)
                  for j in range(_K)]
            for j in range(_K):
                gd[j].wait()
            for j in range(_K):
                pltpu.async_copy(rows[b2].at[j], acc.at[dsts[b3].at[j]],
                                 ssem, add=True)

        fire_idx(0, 0, 0)

        def step(gs, carry):
            for u in range(_UNROLL):
                one_chunk(gs * _UNROLL + u, u)
            return carry

        lax.fori_loop(0, cpt // _UNROLL, step, 0)
        for u in range(2):
            drain_scat(u)
        plsc.subcore_barrier()

        @pl.when(cid == 0)
        def _():
            pltpu.sync_copy(acc.at[nsl], p0.at[nsl])

        @pl.when(cid == 1)
        def _():
            pltpu.sync_copy(acc.at[nsl], p1.at[nsl])

    return pl.kernel(
        body,
        out_type=(jax.ShapeDtypeStruct((n_pad, feat), f32),
                  jax.ShapeDtypeStruct((n_pad, feat), f32)),
        mesh=mesh,
        scratch_types=(pltpu.VMEM_SHARED((n_pad, feat), f32),
                       pltpu.VMEM((_K, _LANES), jnp.int32),
                       pltpu.VMEM((_K, _LANES), jnp.int32),
                       pltpu.VMEM((_K, _LANES), jnp.int32),
                       pltpu.VMEM((_K, _LANES), jnp.int32),
                       pltpu.VMEM((_K, _LANES), jnp.int32),
                       pltpu.VMEM((_K, _LANES, feat), f32),
                       pltpu.VMEM((_K, _LANES, feat), f32),
                       pltpu.SemaphoreType.DMA,
                       pltpu.SemaphoreType.DMA,
                       pltpu.SemaphoreType.DMA),
        compiler_params=pltpu.CompilerParams(use_tc_tiling_on_sc=False))


@functools.lru_cache(maxsize=None)
def _deg_make(n_pad, rpt_n, feat, cpt0, cpt1):
    f32 = jnp.float32
    mesh = plsc.VectorSubcoreMesh(core_axis_name="c", subcore_axis_name="s",
                                  num_cores=_NC, num_subcores=_NS)
    rpt_e0, rpt_e1 = cpt0 * _K, cpt1 * _K

    def body(dst2, zz, ones, q0, q1, accd, dst_a, dst_b, dst_c, ones_v,
             lsem, ssem):
        cid = lax.axis_index("c")
        sid = lax.axis_index("s")
        dsts = (dst_a, dst_b, dst_c)
        nsl = pl.ds(sid * rpt_n, rpt_n)
        pltpu.sync_copy(zz.at[nsl], accd.at[nsl])
        pltpu.sync_copy(ones, ones_v)
        plsc.subcore_barrier()
        row0 = jnp.where(cid == 0, sid * rpt_e0,
                         _NS * rpt_e0 + sid * rpt_e1)
        cpt = jnp.where(cid == 0, cpt0, cpt1)

        def fire_idx(g, b3):
            base = row0 + g * _K
            pltpu.async_copy(dst2.at[pl.ds(base, _K)], dsts[b3], lsem)

        def drain_scat():
            for j in range(_K):
                pltpu.make_async_copy(zz.at[pl.ds(0, _LANES)],
                                      ones_v, ssem).wait()

        def one_chunk(g, u):
            b3 = u % 3

            @pl.when(g >= 2)
            def _():
                drain_scat()

            pltpu.make_async_copy(dst2.at[pl.ds(0, _K)], dsts[b3], lsem).wait()

            @pl.when(g + 1 < cpt)
            def _():
                fire_idx(g + 1, (u + 1) % 3)

            for j in range(_K):
                pltpu.async_copy(ones_v, accd.at[dsts[b3].at[j]],
                                 ssem, add=True)

        fire_idx(0, 0)

        def step(gs, carry):
            for u in range(_UNROLL):
                one_chunk(gs * _UNROLL + u, u)
            return carry

        lax.fori_loop(0, cpt // _UNROLL, step, 0)
        for _u in range(2):
            drain_scat()
        plsc.subcore_barrier()

        @pl.when(cid == 0)
        def _():
            pltpu.sync_copy(accd.at[nsl], q0.at[nsl])

        @pl.when(cid == 1)
        def _():
            pltpu.sync_copy(accd.at[nsl], q1.at[nsl])

    return pl.kernel(
        body,
        out_type=(jax.ShapeDtypeStruct((n_pad, feat), f32),
                  jax.ShapeDtypeStruct((n_pad, feat), f32)),
        mesh=mesh,
        scratch_types=(pltpu.VMEM_SHARED((n_pad, feat), f32),
                       pltpu.VMEM((_K, _LANES), jnp.int32),
                       pltpu.VMEM((_K, _LANES), jnp.int32),
                       pltpu.VMEM((_K, _LANES), jnp.int32),
                       pltpu.VMEM((_LANES, feat), f32),
                       pltpu.SemaphoreType.DMA,
                       pltpu.SemaphoreType.DMA),
        compiler_params=pltpu.CompilerParams(use_tc_tiling_on_sc=False))


def _prep_body(x_ref, w_ref, q0_ref, q1_ref, hn_ref, d_ref):
    d = lax.rsqrt(q0_ref[...] + q1_ref[...] + 1.0)
    d_ref[...] = d
    hn_ref[...] = jnp.dot(x_ref[...], w_ref[...],
                          preferred_element_type=jnp.float32) * d


def _mid_body(p0_ref, p1_ref, hn_ref, d_ref, b_ref, w_ref, o_ref):
    d = d_ref[...]
    t = (p0_ref[...] + p1_ref[...] + hn_ref[...]) * d + b_ref[...]
    t = jnp.maximum(t, 0.0)
    o_ref[...] = jnp.dot(t, w_ref[...], preferred_element_type=jnp.float32) * d


def _fin_body(p0_ref, p1_ref, hn_ref, d_ref, b_ref, o_ref):
    o_ref[...] = (p0_ref[...] + p1_ref[...] + hn_ref[...]) * d_ref[...] + b_ref[...]


def _row_spec(blk):
    return pl.BlockSpec((blk, _PK * _F), lambda i: (i, 0))


def _full_spec(shape):
    return pl.BlockSpec(shape, lambda i: (0, 0))


def _tc_call(body, rows_pk, in_arrays, in_specs, n_out):
    blk = rows_pk // 4
    oshape = jax.ShapeDtypeStruct((rows_pk, _PK * _F), jnp.float32)
    out_shape = [oshape] * n_out if n_out > 1 else oshape
    out_specs = [_row_spec(blk)] * n_out if n_out > 1 else _row_spec(blk)
    return pl.pallas_call(
        body,
        grid=(4,),
        in_specs=in_specs,
        out_specs=out_specs,
        out_shape=out_shape)(*in_arrays)


def kernel(x, edge_index, batch_index, W1, b1, W2, b2, W3, b3):
    f32 = jnp.float32
    n, seq = x.shape
    e = edge_index.shape[1]
    emb = W1.shape[1]
    out_d = W3.shape[1]

    n_pad = _cdiv(n + 1, 1024) * 1024
    rpt_n = n_pad // _NS
    rows_pk = n_pad // _PK
    ct = _cdiv(_cdiv(e, _NS * _K * _LANES), 2 * _UNROLL) * 2 * _UNROLL
    seg_c0 = int(round(ct * 0.72 / _UNROLL)) * _UNROLL
    deg_c0 = int(round(ct * 0.60 / _UNROLL)) * _UNROLL
    rows2d = _NS * _K * ct
    pad = rows2d * _LANES - e

    src2 = jnp.concatenate(
        [edge_index[0], jnp.zeros((pad,), jnp.int32)]).reshape(rows2d, _LANES)
    dst2 = jnp.concatenate(
        [edge_index[1], jnp.full((pad,), n, jnp.int32)]).reshape(rows2d, _LANES)

    eye8 = jnp.eye(_PK, dtype=f32)
    xp = jnp.pad(x, ((0, n_pad - n), (0, _F - seq))).reshape(rows_pk, _PK * _F)
    W1b = jnp.kron(eye8, jnp.pad(W1, ((0, _F - seq), (0, _F - emb))))
    W2b = jnp.kron(eye8, jnp.pad(W2, ((0, _F - emb), (0, _F - emb))))
    W3b = jnp.kron(eye8, jnp.pad(W3, ((0, _F - emb), (0, _F - out_d))))
    b1b = jnp.tile(jnp.pad(b1, (0, _F - emb)), _PK).reshape(1, _PK * _F)
    b2b = jnp.tile(jnp.pad(b2, (0, _F - emb)), _PK).reshape(1, _PK * _F)
    b3b = jnp.tile(jnp.pad(b3, (0, _F - out_d)), _PK).reshape(1, _PK * _F)

    zz = jnp.zeros((n_pad, _F), f32)
    ones = jnp.ones((_LANES, _F), f32)

    deg_fn = _deg_make(n_pad, rpt_n, _F, deg_c0, ct - deg_c0)
    seg_fn = _seg_make(n_pad, rpt_n, _F, seg_c0, ct - seg_c0)

    def pk(a):
        return a.reshape(rows_pk, _PK * _F)

    def unpk(a):
        return a.reshape(n_pad, _F)

    dq0, dq1 = deg_fn(dst2, zz, ones)

    hn1, dpk = _tc_call(_prep_body, rows_pk, (xp, W1b, pk(dq0), pk(dq1)),
                        [_row_spec(rows_pk // 4), _full_spec((_PK * _F, _PK * _F)),
                         _row_spec(rows_pk // 4), _row_spec(rows_pk // 4)], 2)

    s0, s1 = seg_fn(unpk(hn1), src2, dst2, zz)
    hn2 = _tc_call(_mid_body, rows_pk, (pk(s0), pk(s1), hn1, dpk, b1b, W2b),
                   [_row_spec(rows_pk // 4)] * 4 +
                   [_full_spec((1, _PK * _F)), _full_spec((_PK * _F, _PK * _F))], 1)

    s0, s1 = seg_fn(unpk(hn2), src2, dst2, zz)
    hn3 = _tc_call(_mid_body, rows_pk, (pk(s0), pk(s1), hn2, dpk, b2b, W3b),
                   [_row_spec(rows_pk // 4)] * 4 +
                   [_full_spec((1, _PK * _F)), _full_spec((_PK * _F, _PK * _F))], 1)

    s0, s1 = seg_fn(unpk(hn3), src2, dst2, zz)
    outp = _tc_call(_fin_body, rows_pk, (pk(s0), pk(s1), hn3, dpk, b3b),
                    [_row_spec(rows_pk // 4)] * 4 +
                    [_full_spec((1, _PK * _F))], 1)

    return unpk(outp)[:n, :out_d]


# speedup vs baseline: 1.2051x; 1.0314x over previous
import functools

import jax
import jax.numpy as jnp
from jax import lax
from jax.experimental import pallas as pl
from jax.experimental.pallas import tpu as pltpu
from jax.experimental.pallas import tpu_sc as plsc

_NC = 2
_NS = 16
_LANES = 128
_K = 6
_UNROLL = 6
_F = 16
_PK = 8


def _cdiv(a, b):
    return -(-a // b)


@functools.lru_cache(maxsize=None)
def _seg_make(n_pad, rpt_n, feat, cpt0, cpt1):
    f32 = jnp.float32
    mesh = plsc.VectorSubcoreMesh(core_axis_name="c", subcore_axis_name="s",
                                  num_cores=_NC, num_subcores=_NS)
    rpt_e0, rpt_e1 = cpt0 * _K, cpt1 * _K

    def body(hn, src2, dst2, zz, p0, p1, acc,
             src_a, src_b, dst_a, dst_b, dst_c, rows_a, rows_b,
             lsem, gsem, ssem):
        cid = lax.axis_index("c")
        sid = lax.axis_index("s")
        srcs = (src_a, src_b)
        dsts = (dst_a, dst_b, dst_c)
        rows = (rows_a, rows_b)
        nsl = pl.ds(sid * rpt_n, rpt_n)
        pltpu.sync_copy(zz.at[nsl], acc.at[nsl])
        plsc.subcore_barrier()
        row0 = jnp.where(cid == 0, sid * rpt_e0,
                         _NS * rpt_e0 + sid * rpt_e1)
        cpt = jnp.where(cid == 0, cpt0, cpt1)

        def fire_idx(g, b2, b3):
            base = row0 + g * _K
            pltpu.async_copy(src2.at[pl.ds(base, _K)], srcs[b2], lsem)
            pltpu.async_copy(dst2.at[pl.ds(base, _K)], dsts[b3], lsem)

        def drain_scat(b2):
            for j in range(_K):
                pltpu.make_async_copy(zz.at[pl.ds(0, _LANES)],
                                      rows[b2].at[j], ssem).wait()

        def one_chunk(g, u):
            b2, b3 = u % 2, u % 3

            @pl.when(g >= 2)
            def _():
                drain_scat(b2)
            pltpu.make_async_copy(src2.at[pl.ds(0, _K)], srcs[b2], lsem).wait()
            pltpu.make_async_copy(src2.at[pl.ds(0, _K)], dsts[b3], lsem).wait()

            @pl.when(g + 1 < cpt)
            def _():
                fire_idx(g + 1, (u + 1) % 2, (u + 1) % 3)

            gd = [pltpu.async_copy(hn.at[srcs[b2].at[j]], rows[b2].at[j], gsem)
                  for j in range(_K)]
            for j in range(_K):
                gd[j].wait()
            for j in range(_K):
                pltpu.async_copy(rows[b2].at[j], acc.at[dsts[b3].at[j]],
                                 ssem, add=True)

        fire_idx(0, 0, 0)

        def step(gs, carry):
            for u in range(_UNROLL):
                one_chunk(gs * _UNROLL + u, u)
            return carry

        lax.fori_loop(0, cpt // _UNROLL, step, 0)
        for u in range(2):
            drain_scat(u)
        plsc.subcore_barrier()

        @pl.when(cid == 0)
        def _():
            pltpu.sync_copy(acc.at[nsl], p0.at[nsl])

        @pl.when(cid == 1)
        def _():
            pltpu.sync_copy(acc.at[nsl], p1.at[nsl])

    return pl.kernel(
        body,
        out_type=(jax.ShapeDtypeStruct((n_pad, feat), f32),
                  jax.ShapeDtypeStruct((n_pad, feat), f32)),
        mesh=mesh,
        scratch_types=(pltpu.VMEM_SHARED((n_pad, feat), f32),
                       pltpu.VMEM((_K, _LANES), jnp.int32),
                       pltpu.VMEM((_K, _LANES), jnp.int32),
                       pltpu.VMEM((_K, _LANES), jnp.int32),
                       pltpu.VMEM((_K, _LANES), jnp.int32),
                       pltpu.VMEM((_K, _LANES), jnp.int32),
                       pltpu.VMEM((_K, _LANES, feat), f32),
                       pltpu.VMEM((_K, _LANES, feat), f32),
                       pltpu.SemaphoreType.DMA,
                       pltpu.SemaphoreType.DMA,
                       pltpu.SemaphoreType.DMA),
        compiler_params=pltpu.CompilerParams(use_tc_tiling_on_sc=False))


@functools.lru_cache(maxsize=None)
def _deg_make(n_pad, rpt_n, feat, cpt0, cpt1):
    f32 = jnp.float32
    mesh = plsc.VectorSubcoreMesh(core_axis_name="c", subcore_axis_name="s",
                                  num_cores=_NC, num_subcores=_NS)
    rpt_e0, rpt_e1 = cpt0 * _K, cpt1 * _K

    def body(dst2, zz, ones, q0, q1, accd, dst_a, dst_b, dst_c, ones_v,
             lsem, ssem):
        cid = lax.axis_index("c")
        sid = lax.axis_index("s")
        dsts = (dst_a, dst_b, dst_c)
        nsl = pl.ds(sid * rpt_n, rpt_n)
        pltpu.sync_copy(zz.at[nsl], accd.at[nsl])
        pltpu.sync_copy(ones, ones_v)
        plsc.subcore_barrier()
        row0 = jnp.where(cid == 0, sid * rpt_e0,
                         _NS * rpt_e0 + sid * rpt_e1)
        cpt = jnp.where(cid == 0, cpt0, cpt1)

        def fire_idx(g, b3):
            base = row0 + g * _K
            pltpu.async_copy(dst2.at[pl.ds(base, _K)], dsts[b3], lsem)

        def drain_scat():
            for j in range(_K):
                pltpu.make_async_copy(zz.at[pl.ds(0, _LANES)],
                                      ones_v, ssem).wait()

        def one_chunk(g, u):
            b3 = u % 3

            @pl.when(g >= 2)
            def _():
                drain_scat()

            pltpu.make_async_copy(dst2.at[pl.ds(0, _K)], dsts[b3], lsem).wait()

            @pl.when(g + 1 < cpt)
            def _():
                fire_idx(g + 1, (u + 1) % 3)

            for j in range(_K):
                pltpu.async_copy(ones_v, accd.at[dsts[b3].at[j]],
                                 ssem, add=True)

        fire_idx(0, 0)

        def step(gs, carry):
            for u in range(_UNROLL):
                one_chunk(gs * _UNROLL + u, u)
            return carry

        lax.fori_loop(0, cpt // _UNROLL, step, 0)
        for _u in range(2):
            drain_scat()
        plsc.subcore_barrier()

        @pl.when(cid == 0)
        def _():
            pltpu.sync_copy(accd.at[nsl], q0.at[nsl])

        @pl.when(cid == 1)
        def _():
            pltpu.sync_copy(accd.at[nsl], q1.at[nsl])

    return pl.kernel(
        body,
        out_type=(jax.ShapeDtypeStruct((n_pad, feat), f32),
                  jax.ShapeDtypeStruct((n_pad, feat), f32)),
        mesh=mesh,
        scratch_types=(pltpu.VMEM_SHARED((n_pad, feat), f32),
                       pltpu.VMEM((_K, _LANES), jnp.int32),
                       pltpu.VMEM((_K, _LANES), jnp.int32),
                       pltpu.VMEM((_K, _LANES), jnp.int32),
                       pltpu.VMEM((_LANES, feat), f32),
                       pltpu.SemaphoreType.DMA,
                       pltpu.SemaphoreType.DMA),
        compiler_params=pltpu.CompilerParams(use_tc_tiling_on_sc=False))


def _prep_body(x_ref, w_ref, q0_ref, q1_ref, hn_ref, d_ref):
    d = lax.rsqrt(q0_ref[...] + q1_ref[...] + 1.0)
    d_ref[...] = d
    hn_ref[...] = jnp.dot(x_ref[...], w_ref[...],
                          preferred_element_type=jnp.float32) * d


def _mid_body(p0_ref, p1_ref, hn_ref, d_ref, b_ref, w_ref, o_ref):
    d = d_ref[...]
    t = (p0_ref[...] + p1_ref[...] + hn_ref[...]) * d + b_ref[...]
    t = jnp.maximum(t, 0.0)
    o_ref[...] = jnp.dot(t, w_ref[...], preferred_element_type=jnp.float32) * d


def _fin_make(out_d):
    def _fin_body(p0_ref, p1_ref, hn_ref, d_ref, b_ref, o_ref):
        t = (p0_ref[...] + p1_ref[...] + hn_ref[...]) * d_ref[...] + b_ref[...]
        blk = t.shape[0]
        o_ref[...] = t.reshape(blk, _PK, _F)[:, :, :out_d].reshape(blk * _PK, out_d)
    return _fin_body


def _row_spec(blk):
    return pl.BlockSpec((blk, _PK * _F), lambda i: (i, 0))


def _full_spec(shape):
    return pl.BlockSpec(shape, lambda i: (0, 0))


def _tc_call(body, rows_pk, in_arrays, in_specs, n_out):
    blk = rows_pk // 4
    oshape = jax.ShapeDtypeStruct((rows_pk, _PK * _F), jnp.float32)
    out_shape = [oshape] * n_out if n_out > 1 else oshape
    out_specs = [_row_spec(blk)] * n_out if n_out > 1 else _row_spec(blk)
    return pl.pallas_call(
        body,
        grid=(4,),
        in_specs=in_specs,
        out_specs=out_specs,
        out_shape=out_shape)(*in_arrays)


def kernel(x, edge_index, batch_index, W1, b1, W2, b2, W3, b3):
    f32 = jnp.float32
    n, seq = x.shape
    e = edge_index.shape[1]
    emb = W1.shape[1]
    out_d = W3.shape[1]

    n_pad = _cdiv(n + 1, 1024) * 1024
    rpt_n = n_pad // _NS
    rows_pk = n_pad // _PK
    ct = _cdiv(_cdiv(e, _NS * _K * _LANES), 2 * _UNROLL) * 2 * _UNROLL
    seg_c0 = int(round(ct * 0.72 / _UNROLL)) * _UNROLL
    deg_c0 = int(round(ct * 0.60 / _UNROLL)) * _UNROLL
    rows2d = _NS * _K * ct
    pad = rows2d * _LANES - e

    src2 = jnp.concatenate(
        [edge_index[0], jnp.zeros((pad,), jnp.int32)]).reshape(rows2d, _LANES)
    dst2 = jnp.concatenate(
        [edge_index[1], jnp.full((pad,), n, jnp.int32)]).reshape(rows2d, _LANES)

    eye8 = jnp.eye(_PK, dtype=f32)
    xp = jnp.pad(x, ((0, n_pad - n), (0, _F - seq))).reshape(rows_pk, _PK * _F)
    W1b = jnp.kron(eye8, jnp.pad(W1, ((0, _F - seq), (0, _F - emb))))
    W2b = jnp.kron(eye8, jnp.pad(W2, ((0, _F - emb), (0, _F - emb))))
    W3b = jnp.kron(eye8, jnp.pad(W3, ((0, _F - emb), (0, _F - out_d))))
    b1b = jnp.tile(jnp.pad(b1, (0, _F - emb)), _PK).reshape(1, _PK * _F)
    b2b = jnp.tile(jnp.pad(b2, (0, _F - emb)), _PK).reshape(1, _PK * _F)
    b3b = jnp.tile(jnp.pad(b3, (0, _F - out_d)), _PK).reshape(1, _PK * _F)

    zz = jnp.zeros((n_pad, _F), f32)
    ones = jnp.ones((_LANES, _F), f32)

    deg_fn = _deg_make(n_pad, rpt_n, _F, deg_c0, ct - deg_c0)
    seg_fn = _seg_make(n_pad, rpt_n, _F, seg_c0, ct - seg_c0)

    def pk(a):
        return a.reshape(rows_pk, _PK * _F)

    def unpk(a):
        return a.reshape(n_pad, _F)

    dq0, dq1 = deg_fn(dst2, zz, ones)

    hn1, dpk = _tc_call(_prep_body, rows_pk, (xp, W1b, pk(dq0), pk(dq1)),
                        [_row_spec(rows_pk // 4), _full_spec((_PK * _F, _PK * _F)),
                         _row_spec(rows_pk // 4), _row_spec(rows_pk // 4)], 2)

    s0, s1 = seg_fn(unpk(hn1), src2, dst2, zz)
    hn2 = _tc_call(_mid_body, rows_pk, (pk(s0), pk(s1), hn1, dpk, b1b, W2b),
                   [_row_spec(rows_pk // 4)] * 4 +
                   [_full_spec((1, _PK * _F)), _full_spec((_PK * _F, _PK * _F))], 1)

    s0, s1 = seg_fn(unpk(hn2), src2, dst2, zz)
    hn3 = _tc_call(_mid_body, rows_pk, (pk(s0), pk(s1), hn2, dpk, b2b, W3b),
                   [_row_spec(rows_pk // 4)] * 4 +
                   [_full_spec((1, _PK * _F)), _full_spec((_PK * _F, _PK * _F))], 1)

    s0, s1 = seg_fn(unpk(hn3), src2, dst2, zz)
    blk = rows_pk // 4
    outp = pl.pallas_call(
        _fin_make(out_d),
        grid=(4,),
        in_specs=[_row_spec(blk)] * 4 + [_full_spec((1, _PK * _F))],
        out_specs=pl.BlockSpec((blk * _PK, out_d), lambda i: (i, 0)),
        out_shape=jax.ShapeDtypeStruct((n, out_d), jnp.float32),
    )(pk(s0), pk(s1), hn3, dpk, b3b)

    return outp
